# baseline (speedup 1.0000x reference)
import functools

import jax
import jax.numpy as jnp
import numpy as np
from jax.experimental import pallas as pl
from jax.experimental.pallas import tpu as pltpu

_PAR = pltpu.CompilerParams(dimension_semantics=("parallel",))

_N = 10000
_IMG_N = 3
_IN_F = 128
_OUT_F = 8
_F32 = jnp.float32


def _encode_body(x_ref, xa_ref, w_ref, o_ref):
    w = w_ref[...]
    u = jnp.dot(x_ref[...], w, preferred_element_type=_F32)
    ua = jnp.dot(xa_ref[...], w, preferred_element_type=_F32)
    o_ref[...] = jnp.concatenate([u, ua], axis=1)


def _spmm1_body(a_ref, b_ref, z24_ref, z_ref, rhsg_ref):
    z = jnp.dot(a_ref[...], b_ref[...], preferred_element_type=_F32)
    z24_ref[...] = z[:, :24]
    z_ref[...] = z
    tm = z.shape[0]
    rhsg_ref[...] = jnp.concatenate(
        [jax.nn.relu(z), jnp.ones((tm, 16), _F32)], axis=1)


def _spmm_body(a_ref, b_ref, o_ref):
    o_ref[...] = jnp.dot(a_ref[...], b_ref[...], preferred_element_type=_F32)


def _epi_body(z_ref, z2_ref, gg_ref, wde_ref, wdisc_ref, s_ref, mpos_ref,
              mneg_ref, db_ref, recs_ref, pos_ref, neg_ref):
    z = z_ref[...]
    ew = jnp.dot(jax.nn.relu(z), wdisc_ref[...], preferred_element_type=_F32)
    gg = gg_ref[...]
    ge = gg[:, :48] / gg[:, 48:49]
    grp = jnp.dot(ge * ge, s_ref[...], preferred_element_type=_F32)
    g = jax.nn.sigmoid(ge / jnp.maximum(jnp.sqrt(grp), 1e-12))
    gp = jnp.concatenate([g[:, :24], g[:, :24]], axis=1)
    ga = jnp.concatenate([g[:, 24:48], g[:, 24:48]], axis=1)
    db = db_ref[0, 0]
    pos6 = jnp.dot(ew * gp, mpos_ref[...], preferred_element_type=_F32) + db
    neg6 = jnp.dot(ew * ga, mneg_ref[...], preferred_element_type=_F32) + db
    z2 = z2_ref[...]
    wde = wde_ref[...]
    for i in range(3):
        pos_ref[:, i, :] = pos6[:, 2 * i:2 * i + 2]
        neg_ref[:, i, :] = neg6[:, 2 * i:2 * i + 2]
        recs_ref[:, i, :] = jnp.dot(
            z2[:, 8 * i:8 * i + 8], wde[8 * i:8 * i + 8, 128 * i:128 * i + 128],
            preferred_element_type=_F32)


def _row_spec(tm, ncols):
    return pl.BlockSpec((tm, ncols), lambda i: (i, 0))


def _full_spec(shape):
    return pl.BlockSpec(shape, lambda i: (0, 0))


@functools.partial(jax.jit, static_argnames=())
def kernel(img, img_a, adj, graph_neigh, en_weight1, de_weight1, disc_w,
           disc_b):
    n = img.shape[0]
    x = img.reshape(n, _IMG_N * _IN_F)
    xa = img_a.reshape(n, _IMG_N * _IN_F)

    wen = jnp.zeros((_IMG_N * _IN_F, _IMG_N * _OUT_F), _F32)
    wde = jnp.zeros((_IMG_N * _OUT_F, _IMG_N * _IN_F), _F32)
    wdisc = jnp.zeros((48, 48), _F32)
    for i in range(_IMG_N):
        wen = wen.at[i * _IN_F:(i + 1) * _IN_F,
                     i * _OUT_F:(i + 1) * _OUT_F].set(en_weight1[:, i, :])
        wde = wde.at[i * _OUT_F:(i + 1) * _OUT_F,
                     i * _IN_F:(i + 1) * _IN_F].set(de_weight1[:, i, :])
    for j in range(6):
        wdisc = wdisc.at[j * 8:(j + 1) * 8, j * 8:(j + 1) * 8].set(disc_w[0])

    s_np = np.kron(np.eye(6, dtype=np.float32), np.ones((8, 8), np.float32))
    mpos_np = np.zeros((48, 6), np.float32)
    mneg_np = np.zeros((48, 6), np.float32)
    for i in range(3):
        mpos_np[8 * i:8 * i + 8, 2 * i] = 1.0
        mpos_np[24 + 8 * i:24 + 8 * i + 8, 2 * i + 1] = 1.0
        mneg_np[24 + 8 * i:24 + 8 * i + 8, 2 * i] = 1.0
        mneg_np[8 * i:8 * i + 8, 2 * i + 1] = 1.0
    s_c = jnp.asarray(s_np)
    mpos_c = jnp.asarray(mpos_np)
    mneg_c = jnp.asarray(mneg_np)

    tm_e = 1000
    rhs1 = pl.pallas_call(
        _encode_body,
        grid=(n // tm_e,),
        in_specs=[_row_spec(tm_e, 384), _row_spec(tm_e, 384),
                  _full_spec((384, 24))],
        out_specs=_row_spec(tm_e, 48),
        out_shape=jax.ShapeDtypeStruct((n, 48), _F32),
        compiler_params=_PAR,
    )(x, xa, wen)

    tm = 200
    z24, z_all, rhsg = pl.pallas_call(
        _spmm1_body,
        grid=(n // tm,),
        in_specs=[_row_spec(tm, n), _full_spec((n, 48))],
        out_specs=[_row_spec(tm, 24), _row_spec(tm, 48), _row_spec(tm, 64)],
        out_shape=[jax.ShapeDtypeStruct((n, 24), _F32),
                   jax.ShapeDtypeStruct((n, 48), _F32),
                   jax.ShapeDtypeStruct((n, 64), _F32)],
        compiler_params=_PAR,
    )(adj, rhs1)

    z2 = pl.pallas_call(
        _spmm_body,
        grid=(n // tm,),
        in_specs=[_row_spec(tm, n), _full_spec((n, 24))],
        out_specs=_row_spec(tm, 24),
        out_shape=jax.ShapeDtypeStruct((n, 24), _F32),
        compiler_params=_PAR,
    )(adj, z24)

    gg = pl.pallas_call(
        _spmm_body,
        grid=(n // tm,),
        in_specs=[_row_spec(tm, n), _full_spec((n, 64))],
        out_specs=_row_spec(tm, 64),
        out_shape=jax.ShapeDtypeStruct((n, 64), _F32),
        compiler_params=_PAR,
    )(graph_neigh, rhsg)

    tm2 = 400
    db2 = disc_b.reshape(1, 1)
    recs, poss, negs = pl.pallas_call(
        _epi_body,
        grid=(n // tm2,),
        in_specs=[_row_spec(tm2, 48), _row_spec(tm2, 24), _row_spec(tm2, 64),
                  _full_spec((24, 384)), _full_spec((48, 48)),
                  _full_spec((48, 48)), _full_spec((48, 6)),
                  _full_spec((48, 6)), _full_spec((1, 1))],
        out_specs=[pl.BlockSpec((tm2, 3, 128), lambda i: (i, 0, 0)),
                   pl.BlockSpec((tm2, 3, 2), lambda i: (i, 0, 0)),
                   pl.BlockSpec((tm2, 3, 2), lambda i: (i, 0, 0))],
        out_shape=[jax.ShapeDtypeStruct((n, 3, 128), _F32),
                   jax.ShapeDtypeStruct((n, 3, 2), _F32),
                   jax.ShapeDtypeStruct((n, 3, 2), _F32)],
        compiler_params=_PAR,
    )(z_all, z2, gg, wde, wdisc, s_c, mpos_c, mneg_c, db2)

    return (z24, recs, poss, negs)


# speedup vs baseline: 1.1123x; 1.1123x over previous
import functools

import jax
import jax.numpy as jnp
import numpy as np
from jax.experimental import pallas as pl
from jax.experimental.pallas import tpu as pltpu

_PAR = pltpu.CompilerParams(dimension_semantics=("parallel",))

_N = 10000
_IMG_N = 3
_IN_F = 128
_OUT_F = 8
_F32 = jnp.float32


def _encode_body(x_ref, xa_ref, w_ref, o_ref):
    w = w_ref[...]
    u = jnp.dot(x_ref[...], w, preferred_element_type=_F32)
    ua = jnp.dot(xa_ref[...], w, preferred_element_type=_F32)
    o_ref[...] = jnp.concatenate([u, ua], axis=1)


def _spmm1_body(a_ref, b_ref, z24_ref, z_ref, rhsg_ref):
    z = jnp.dot(a_ref[...], b_ref[...], preferred_element_type=_F32)
    z24_ref[...] = z[:, :24]
    z_ref[...] = z
    tm = z.shape[0]
    rhsg_ref[...] = jnp.concatenate(
        [jax.nn.relu(z), jnp.ones((tm, 16), _F32)], axis=1)


def _spmm_body(a_ref, b_ref, o_ref):
    o_ref[...] = jnp.dot(a_ref[...], b_ref[...], preferred_element_type=_F32)


def _epi_body(z_ref, z2_ref, gg_ref, wde_ref, wdisc_ref, s_ref, mpos_ref,
              mneg_ref, db_ref, recs_ref, pos_ref, neg_ref):
    z = z_ref[...]
    ew = jnp.dot(jax.nn.relu(z), wdisc_ref[...], preferred_element_type=_F32)
    gg = gg_ref[...]
    ge = gg[:, :48] / gg[:, 48:49]
    grp = jnp.dot(ge * ge, s_ref[...], preferred_element_type=_F32)
    g = jax.nn.sigmoid(ge / jnp.maximum(jnp.sqrt(grp), 1e-12))
    gp = jnp.concatenate([g[:, :24], g[:, :24]], axis=1)
    ga = jnp.concatenate([g[:, 24:48], g[:, 24:48]], axis=1)
    db = db_ref[0, 0]
    pos6 = jnp.dot(ew * gp, mpos_ref[...], preferred_element_type=_F32) + db
    neg6 = jnp.dot(ew * ga, mneg_ref[...], preferred_element_type=_F32) + db
    pos_ref[...] = pos6
    neg_ref[...] = neg6
    z2 = z2_ref[...]
    wde = wde_ref[...]
    for i in range(3):
        recs_ref[:, i, :] = jnp.dot(
            z2[:, 8 * i:8 * i + 8], wde[8 * i:8 * i + 8, 128 * i:128 * i + 128],
            preferred_element_type=_F32)


def _row_spec(tm, ncols):
    return pl.BlockSpec((tm, ncols), lambda i: (i, 0))


def _full_spec(shape):
    return pl.BlockSpec(shape, lambda i: (0, 0))


@functools.partial(jax.jit, static_argnames=())
def kernel(img, img_a, adj, graph_neigh, en_weight1, de_weight1, disc_w,
           disc_b):
    n = img.shape[0]
    x = img.reshape(n, _IMG_N * _IN_F)
    xa = img_a.reshape(n, _IMG_N * _IN_F)

    wen = jnp.zeros((_IMG_N * _IN_F, _IMG_N * _OUT_F), _F32)
    wde = jnp.zeros((_IMG_N * _OUT_F, _IMG_N * _IN_F), _F32)
    wdisc = jnp.zeros((48, 48), _F32)
    for i in range(_IMG_N):
        wen = wen.at[i * _IN_F:(i + 1) * _IN_F,
                     i * _OUT_F:(i + 1) * _OUT_F].set(en_weight1[:, i, :])
        wde = wde.at[i * _OUT_F:(i + 1) * _OUT_F,
                     i * _IN_F:(i + 1) * _IN_F].set(de_weight1[:, i, :])
    for j in range(6):
        wdisc = wdisc.at[j * 8:(j + 1) * 8, j * 8:(j + 1) * 8].set(disc_w[0])

    s_np = np.kron(np.eye(6, dtype=np.float32), np.ones((8, 8), np.float32))
    mpos_np = np.zeros((48, 6), np.float32)
    mneg_np = np.zeros((48, 6), np.float32)
    for i in range(3):
        mpos_np[8 * i:8 * i + 8, 2 * i] = 1.0
        mpos_np[24 + 8 * i:24 + 8 * i + 8, 2 * i + 1] = 1.0
        mneg_np[24 + 8 * i:24 + 8 * i + 8, 2 * i] = 1.0
        mneg_np[8 * i:8 * i + 8, 2 * i + 1] = 1.0
    s_c = jnp.asarray(s_np)
    mpos_c = jnp.asarray(mpos_np)
    mneg_c = jnp.asarray(mneg_np)

    tm_e = 1000
    rhs1 = pl.pallas_call(
        _encode_body,
        grid=(n // tm_e,),
        in_specs=[_row_spec(tm_e, 384), _row_spec(tm_e, 384),
                  _full_spec((384, 24))],
        out_specs=_row_spec(tm_e, 48),
        out_shape=jax.ShapeDtypeStruct((n, 48), _F32),
        compiler_params=_PAR,
    )(x, xa, wen)

    tm = 200
    z24, z_all, rhsg = pl.pallas_call(
        _spmm1_body,
        grid=(n // tm,),
        in_specs=[_row_spec(tm, n), _full_spec((n, 48))],
        out_specs=[_row_spec(tm, 24), _row_spec(tm, 48), _row_spec(tm, 64)],
        out_shape=[jax.ShapeDtypeStruct((n, 24), _F32),
                   jax.ShapeDtypeStruct((n, 48), _F32),
                   jax.ShapeDtypeStruct((n, 64), _F32)],
        compiler_params=_PAR,
    )(adj, rhs1)

    z2 = pl.pallas_call(
        _spmm_body,
        grid=(n // tm,),
        in_specs=[_row_spec(tm, n), _full_spec((n, 24))],
        out_specs=_row_spec(tm, 24),
        out_shape=jax.ShapeDtypeStruct((n, 24), _F32),
        compiler_params=_PAR,
    )(adj, z24)

    gg = pl.pallas_call(
        _spmm_body,
        grid=(n // tm,),
        in_specs=[_row_spec(tm, n), _full_spec((n, 64))],
        out_specs=_row_spec(tm, 64),
        out_shape=jax.ShapeDtypeStruct((n, 64), _F32),
        compiler_params=_PAR,
    )(graph_neigh, rhsg)

    tm2 = 400
    db2 = disc_b.reshape(1, 1)
    recs, pos6, neg6 = pl.pallas_call(
        _epi_body,
        grid=(n // tm2,),
        in_specs=[_row_spec(tm2, 48), _row_spec(tm2, 24), _row_spec(tm2, 64),
                  _full_spec((24, 384)), _full_spec((48, 48)),
                  _full_spec((48, 48)), _full_spec((48, 6)),
                  _full_spec((48, 6)), _full_spec((1, 1))],
        out_specs=[pl.BlockSpec((tm2, 3, 128), lambda i: (i, 0, 0)),
                   _row_spec(tm2, 6), _row_spec(tm2, 6)],
        out_shape=[jax.ShapeDtypeStruct((n, 3, 128), _F32),
                   jax.ShapeDtypeStruct((n, 6), _F32),
                   jax.ShapeDtypeStruct((n, 6), _F32)],
        compiler_params=_PAR,
    )(z_all, z2, gg, wde, wdisc, s_c, mpos_c, mneg_c, db2)

    return (z24, recs, pos6.reshape(n, _IMG_N, 2), neg6.reshape(n, _IMG_N, 2))


# speedup vs baseline: 1.1328x; 1.0184x over previous
import functools

import jax
import jax.numpy as jnp
import numpy as np
from jax.experimental import pallas as pl
from jax.experimental.pallas import tpu as pltpu

_PAR = pltpu.CompilerParams(dimension_semantics=("parallel",))

_IMG_N = 3
_IN_F = 128
_OUT_F = 8
_F32 = jnp.float32


def _encode_body(x_ref, xa_ref, w_ref, o_ref):
    parts = []
    for src in (x_ref, xa_ref):
        for i in range(_IMG_N):
            parts.append(jnp.dot(src[:, i, :], w_ref[:, i, :],
                                 preferred_element_type=_F32))
    o_ref[...] = jnp.concatenate(parts, axis=1)


def _pass_a_body(a_ref, b_ref, z24_ref, z_ref, rhsg_ref):
    z = jnp.dot(a_ref[...], b_ref[...], preferred_element_type=_F32)
    z24_ref[...] = z[:, :24]
    z_ref[...] = z
    tm = z.shape[0]
    rhsg_ref[...] = jnp.concatenate(
        [jax.nn.relu(z), jnp.ones((tm, 16), _F32)], axis=1)


def _pass_b_body(a_ref, rhsg_ref, z_ref, dw_ref, s_ref, mpos_ref, mneg_ref,
                 db_ref, pos_ref, neg_ref):
    gg = jnp.dot(a_ref[...], rhsg_ref[...], preferred_element_type=_F32)
    z = z_ref[...]
    dw = dw_ref[0]
    ew = jnp.concatenate(
        [jnp.dot(jax.nn.relu(z[:, 8 * j:8 * j + 8]), dw,
                 preferred_element_type=_F32) for j in range(6)], axis=1)
    ge = gg[:, :48] / gg[:, 48:49]
    grp = jnp.dot(ge * ge, s_ref[...], preferred_element_type=_F32)
    g = jax.nn.sigmoid(ge / jnp.maximum(jnp.sqrt(grp), 1e-12))
    gp = jnp.concatenate([g[:, :24], g[:, :24]], axis=1)
    ga = jnp.concatenate([g[:, 24:48], g[:, 24:48]], axis=1)
    db = db_ref[0, 0]
    pos_ref[...] = jnp.dot(ew * gp, mpos_ref[...],
                           preferred_element_type=_F32) + db
    neg_ref[...] = jnp.dot(ew * ga, mneg_ref[...],
                           preferred_element_type=_F32) + db


def _pass_c_body(a_ref, b_ref, dew_ref, recs_ref):
    z2 = jnp.dot(a_ref[...], b_ref[...], preferred_element_type=_F32)
    for i in range(_IMG_N):
        recs_ref[:, i, :] = jnp.dot(z2[:, 8 * i:8 * i + 8], dew_ref[:, i, :],
                                    preferred_element_type=_F32)


def _row_spec(tm, ncols):
    return pl.BlockSpec((tm, ncols), lambda i: (i, 0))


def _full_spec(shape):
    nz = (0,) * len(shape)
    return pl.BlockSpec(shape, lambda i, _nz=nz: _nz)


@functools.partial(jax.jit, static_argnames=())
def kernel(img, img_a, adj, graph_neigh, en_weight1, de_weight1, disc_w,
           disc_b):
    n = img.shape[0]

    s_np = np.kron(np.eye(6, dtype=np.float32), np.ones((8, 8), np.float32))
    mpos_np = np.zeros((48, 6), np.float32)
    mneg_np = np.zeros((48, 6), np.float32)
    for i in range(3):
        mpos_np[8 * i:8 * i + 8, 2 * i] = 1.0
        mpos_np[24 + 8 * i:24 + 8 * i + 8, 2 * i + 1] = 1.0
        mneg_np[24 + 8 * i:24 + 8 * i + 8, 2 * i] = 1.0
        mneg_np[8 * i:8 * i + 8, 2 * i + 1] = 1.0
    s_c = jnp.asarray(s_np)
    mpos_c = jnp.asarray(mpos_np)
    mneg_c = jnp.asarray(mneg_np)
    db2 = disc_b.reshape(1, 1)

    tm_e = 1000
    rhs1 = pl.pallas_call(
        _encode_body,
        grid=(n // tm_e,),
        in_specs=[pl.BlockSpec((tm_e, _IMG_N, _IN_F), lambda i: (i, 0, 0)),
                  pl.BlockSpec((tm_e, _IMG_N, _IN_F), lambda i: (i, 0, 0)),
                  _full_spec((_IN_F, _IMG_N, _OUT_F))],
        out_specs=_row_spec(tm_e, 48),
        out_shape=jax.ShapeDtypeStruct((n, 48), _F32),
        compiler_params=_PAR,
    )(img, img_a, en_weight1)

    tm = 400
    z24, z_all, rhsg = pl.pallas_call(
        _pass_a_body,
        grid=(n // tm,),
        in_specs=[_row_spec(tm, n), _full_spec((n, 48))],
        out_specs=[_row_spec(tm, 24), _row_spec(tm, 48), _row_spec(tm, 64)],
        out_shape=[jax.ShapeDtypeStruct((n, 24), _F32),
                   jax.ShapeDtypeStruct((n, 48), _F32),
                   jax.ShapeDtypeStruct((n, 64), _F32)],
        compiler_params=_PAR,
    )(adj, rhs1)

    pos6, neg6 = pl.pallas_call(
        _pass_b_body,
        grid=(n // tm,),
        in_specs=[_row_spec(tm, n), _full_spec((n, 64)), _row_spec(tm, 48),
                  pl.BlockSpec((1, 8, 8), lambda i: (0, 0, 0)),
                  _full_spec((48, 48)), _full_spec((48, 6)),
                  _full_spec((48, 6)), _full_spec((1, 1))],
        out_specs=[_row_spec(tm, 6), _row_spec(tm, 6)],
        out_shape=[jax.ShapeDtypeStruct((n, 6), _F32),
                   jax.ShapeDtypeStruct((n, 6), _F32)],
        compiler_params=_PAR,
    )(graph_neigh, rhsg, z_all, disc_w, s_c, mpos_c, mneg_c, db2)

    recs = pl.pallas_call(
        _pass_c_body,
        grid=(n // tm,),
        in_specs=[_row_spec(tm, n), _full_spec((n, 24)),
                  pl.BlockSpec((_OUT_F, _IMG_N, _IN_F), lambda i: (0, 0, 0))],
        out_specs=pl.BlockSpec((tm, _IMG_N, _IN_F), lambda i: (i, 0, 0)),
        out_shape=jax.ShapeDtypeStruct((n, _IMG_N, _IN_F), _F32),
        compiler_params=_PAR,
    )(adj, z24, de_weight1)

    return (z24, recs, pos6.reshape(n, _IMG_N, 2), neg6.reshape(n, _IMG_N, 2))


# speedup vs baseline: 1.1505x; 1.0156x over previous
import functools

import jax
import jax.numpy as jnp
import numpy as np
from jax.experimental import pallas as pl
from jax.experimental.pallas import tpu as pltpu

_PAR = pltpu.CompilerParams(dimension_semantics=("parallel",))

_IMG_N = 3
_IN_F = 128
_OUT_F = 8
_F32 = jnp.float32


def _encode_body(x_ref, xa_ref, w_ref, o_ref):
    parts = []
    for src in (x_ref, xa_ref):
        for i in range(_IMG_N):
            parts.append(jnp.dot(src[:, _IN_F * i:_IN_F * (i + 1)],
                                 w_ref[:, i, :], preferred_element_type=_F32))
    o_ref[...] = jnp.concatenate(parts, axis=1)


def _pass_a_body(a_ref, b_ref, z24_ref, score_ref, z_ref, rhsg_ref):
    z = jnp.dot(a_ref[...], b_ref[...], preferred_element_type=_F32)
    z24_ref[...] = z[:, :24]
    score_ref[...] = z[:, :24]
    z_ref[...] = z
    tm = z.shape[0]
    rhsg_ref[...] = jnp.concatenate(
        [jax.nn.relu(z), jnp.ones((tm, 16), _F32)], axis=1)


def _pass_b_body(a_ref, rhsg_ref, z_ref, dw_ref, s_ref, mpos_ref, mneg_ref,
                 db_ref, pos_ref, neg_ref):
    gg = jnp.dot(a_ref[...], rhsg_ref[...], preferred_element_type=_F32)
    z = z_ref[...]
    dw = dw_ref[0]
    ew = jnp.concatenate(
        [jnp.dot(jax.nn.relu(z[:, 8 * j:8 * j + 8]), dw,
                 preferred_element_type=_F32) for j in range(6)], axis=1)
    ge = gg[:, :48] / gg[:, 48:49]
    grp = jnp.dot(ge * ge, s_ref[...], preferred_element_type=_F32)
    g = jax.nn.sigmoid(ge / jnp.maximum(jnp.sqrt(grp), 1e-12))
    gp = jnp.concatenate([g[:, :24], g[:, :24]], axis=1)
    ga = jnp.concatenate([g[:, 24:48], g[:, 24:48]], axis=1)
    db = db_ref[0, 0]
    pos_ref[...] = jnp.dot(ew * gp, mpos_ref[...],
                           preferred_element_type=_F32) + db
    neg_ref[...] = jnp.dot(ew * ga, mneg_ref[...],
                           preferred_element_type=_F32) + db


def _pass_c_body(a_ref, b_ref, dew_ref, recs_ref):
    z2 = jnp.dot(a_ref[...], b_ref[...], preferred_element_type=_F32)
    for i in range(_IMG_N):
        recs_ref[:, i, :] = jnp.dot(z2[:, 8 * i:8 * i + 8], dew_ref[:, i, :],
                                    preferred_element_type=_F32)


def _row_spec(tm, ncols):
    return pl.BlockSpec((tm, ncols), lambda i: (i, 0))


def _full_spec(shape):
    nz = (0,) * len(shape)
    return pl.BlockSpec(shape, lambda i, _nz=nz: _nz)


@functools.partial(jax.jit, static_argnames=())
def kernel(img, img_a, adj, graph_neigh, en_weight1, de_weight1, disc_w,
           disc_b):
    n = img.shape[0]

    s_np = np.kron(np.eye(6, dtype=np.float32), np.ones((8, 8), np.float32))
    mpos_np = np.zeros((48, 6), np.float32)
    mneg_np = np.zeros((48, 6), np.float32)
    for i in range(3):
        mpos_np[8 * i:8 * i + 8, 2 * i] = 1.0
        mpos_np[24 + 8 * i:24 + 8 * i + 8, 2 * i + 1] = 1.0
        mneg_np[24 + 8 * i:24 + 8 * i + 8, 2 * i] = 1.0
        mneg_np[8 * i:8 * i + 8, 2 * i + 1] = 1.0
    s_c = jnp.asarray(s_np)
    mpos_c = jnp.asarray(mpos_np)
    mneg_c = jnp.asarray(mneg_np)
    db2 = disc_b.reshape(1, 1)

    x = img.reshape(n, _IMG_N * _IN_F)
    xa = img_a.reshape(n, _IMG_N * _IN_F)
    tm_e = 1000
    rhs1 = pl.pallas_call(
        _encode_body,
        grid=(n // tm_e,),
        in_specs=[_row_spec(tm_e, _IMG_N * _IN_F),
                  _row_spec(tm_e, _IMG_N * _IN_F),
                  _full_spec((_IN_F, _IMG_N, _OUT_F))],
        out_specs=_row_spec(tm_e, 48),
        out_shape=jax.ShapeDtypeStruct((n, 48), _F32),
        compiler_params=_PAR,
    )(x, xa, en_weight1)

    tm = 400
    z24, score, z_all, rhsg = pl.pallas_call(
        _pass_a_body,
        grid=(n // tm,),
        in_specs=[_row_spec(tm, n), _full_spec((n, 48))],
        out_specs=[_row_spec(tm, 24), _row_spec(tm, 24), _row_spec(tm, 48),
                   _row_spec(tm, 64)],
        out_shape=[jax.ShapeDtypeStruct((n, 24), _F32),
                   jax.ShapeDtypeStruct((n, 24), _F32),
                   jax.ShapeDtypeStruct((n, 48), _F32),
                   jax.ShapeDtypeStruct((n, 64), _F32)],
        compiler_params=_PAR,
    )(adj, rhs1)

    pos6, neg6 = pl.pallas_call(
        _pass_b_body,
        grid=(n // tm,),
        in_specs=[_row_spec(tm, n), _full_spec((n, 64)), _row_spec(tm, 48),
                  pl.BlockSpec((1, 8, 8), lambda i: (0, 0, 0)),
                  _full_spec((48, 48)), _full_spec((48, 6)),
                  _full_spec((48, 6)), _full_spec((1, 1))],
        out_specs=[_row_spec(tm, 6), _row_spec(tm, 6)],
        out_shape=[jax.ShapeDtypeStruct((n, 6), _F32),
                   jax.ShapeDtypeStruct((n, 6), _F32)],
        compiler_params=_PAR,
    )(graph_neigh, rhsg, z_all, disc_w, s_c, mpos_c, mneg_c, db2)

    recs = pl.pallas_call(
        _pass_c_body,
        grid=(n // tm,),
        in_specs=[_row_spec(tm, n), _full_spec((n, 24)),
                  pl.BlockSpec((_OUT_F, _IMG_N, _IN_F), lambda i: (0, 0, 0))],
        out_specs=pl.BlockSpec((tm, _IMG_N, _IN_F), lambda i: (i, 0, 0)),
        out_shape=jax.ShapeDtypeStruct((n, _IMG_N, _IN_F), _F32),
        compiler_params=_PAR,
    )(adj, z24, de_weight1)

    return (score, recs, pos6.reshape(n, _IMG_N, 2),
            neg6.reshape(n, _IMG_N, 2))
